# baseline (speedup 1.0000x reference)
import functools

import jax
import jax.numpy as jnp
from jax import lax
from jax.experimental import pallas as pl
from jax.experimental.pallas import tpu as pltpu
from jax.experimental.pallas import tpu_sc as plsc

N_NODES = 1000000
N_RELATIONS = 1000
EMBED_DIM = 32
BATCH = 16384

_INFO = plsc.get_sparse_core_info()
_NC = _INFO.num_cores
_NS = _INFO.num_subcores
_NW = _NC * _NS
_L = _INFO.num_lanes

_B_PER_W = BATCH // _NW
_CHUNK = 256
_N_CHUNKS = _B_PER_W // _CHUNK
_GROUPS = _CHUNK // _L
_SUB = 8
_NBLK = EMBED_DIM // _SUB
_WIN = 128


def _body(heads_hbm, rels_hbm, tails_hbm,
          nr_hbm, ni_hbm, rTr_hbm, rTi_hbm,
          out_hbm,
          h_idx, r_idx, t_idx,
          srT, siT, trT, tiT,
          relr_v, reli_v,
          out_v, sem, rsem):
    wid = lax.axis_index("s") * _NC + lax.axis_index("c")
    base = wid * _B_PER_W

    cr = pltpu.async_copy(rTr_hbm, relr_v, rsem)
    ci = pltpu.async_copy(rTi_hbm, reli_v, rsem)

    pltpu.sync_copy(heads_hbm.at[pl.ds(base, _B_PER_W)], h_idx)
    pltpu.sync_copy(rels_hbm.at[pl.ds(base, _B_PER_W)], r_idx)
    pltpu.sync_copy(tails_hbm.at[pl.ds(base, _B_PER_W)], t_idx)

    cr.wait()
    ci.wait()

    for chunk in range(_N_CHUNKS):
        off = chunk * _CHUNK

        def issue(g, carry):
            goff = g * _L
            h_ids = h_idx[pl.ds(off + goff, _L)]
            t_ids = t_idx[pl.ds(off + goff, _L)]
            for c in range(EMBED_DIM):
                dpos = pl.ds(c * _CHUNK + goff, _L)
                pltpu.async_copy(nr_hbm.at[c].at[h_ids], srT.at[dpos], sem)
                pltpu.async_copy(ni_hbm.at[c].at[h_ids], siT.at[dpos], sem)
                pltpu.async_copy(nr_hbm.at[c].at[t_ids], trT.at[dpos], sem)
                pltpu.async_copy(ni_hbm.at[c].at[t_ids], tiT.at[dpos], sem)
            return carry

        lax.fori_loop(0, _GROUPS, issue, 0)

        nelems = EMBED_DIM * _CHUNK
        dummy = nr_hbm.at[0].at[pl.ds(0, nelems)]
        pltpu.make_async_copy(dummy, srT, sem).wait()
        pltpu.make_async_copy(dummy, siT, sem).wait()
        pltpu.make_async_copy(dummy, trT, sem).wait()
        pltpu.make_async_copy(dummy, tiT, sem).wait()

        def compute(g, carry):
            goff = g * _L
            rel_ids = r_idx[pl.ds(off + goff, _L)]
            phi = jnp.zeros((_L,), jnp.float32)
            for c in range(EMBED_DIM):
                dpos = pl.ds(c * _CHUNK + goff, _L)
                flat_ids = rel_ids + (c * N_RELATIONS)
                sr_c = srT[dpos]
                si_c = siT[dpos]
                tr_c = trT[dpos]
                ti_c = tiT[dpos]
                rr_c = plsc.load_gather(relr_v, [flat_ids])
                ri_c = plsc.load_gather(reli_v, [flat_ids])
                phi = phi + rr_c * (sr_c * tr_c + si_c * ti_c)
                phi = phi + ri_c * (sr_c * ti_c - si_c * tr_c)
            out_v[pl.ds(off + goff, _L)] = phi
            return carry

        lax.fori_loop(0, _GROUPS, compute, 0)

    pltpu.sync_copy(out_v, out_hbm.at[pl.ds(base, _B_PER_W)])


@jax.jit
def kernel(heads, rels, tails, node_r, node_i, rel_r, rel_i):
    mesh = plsc.VectorSubcoreMesh(core_axis_name="c", subcore_axis_name="s")
    f = functools.partial(
        pl.kernel,
        out_type=jax.ShapeDtypeStruct((BATCH,), jnp.float32),
        mesh=mesh,
        compiler_params=pltpu.CompilerParams(
            use_tc_tiling_on_sc=False, needs_layout_passes=False),
        scratch_types=[
            pltpu.VMEM((_B_PER_W,), jnp.int32),
            pltpu.VMEM((_B_PER_W,), jnp.int32),
            pltpu.VMEM((_B_PER_W,), jnp.int32),
            pltpu.VMEM((EMBED_DIM * _CHUNK,), jnp.float32),
            pltpu.VMEM((EMBED_DIM * _CHUNK,), jnp.float32),
            pltpu.VMEM((EMBED_DIM * _CHUNK,), jnp.float32),
            pltpu.VMEM((EMBED_DIM * _CHUNK,), jnp.float32),
            pltpu.VMEM((EMBED_DIM * N_RELATIONS,), jnp.float32),
            pltpu.VMEM((EMBED_DIM * N_RELATIONS,), jnp.float32),
            pltpu.VMEM((_B_PER_W,), jnp.float32),
            pltpu.SemaphoreType.DMA,
            pltpu.SemaphoreType.DMA,
        ],
    )(_body)
    return f(heads, rels, tails, node_r.T, node_i.T,
             rel_r.T.reshape(EMBED_DIM * N_RELATIONS),
             rel_i.T.reshape(EMBED_DIM * N_RELATIONS))
